# Initial kernel scaffold; baseline (speedup 1.0000x reference)
import functools

import jax
import jax.numpy as jnp
from jax import lax
from jax.experimental import pallas as pl
from jax.experimental.pallas import tpu as pltpu
from jax.experimental.pallas import tpu_sc as plsc

NC = 2
NS = 16
L = 16
CHUNK = 128


def _sc_mesh():
    return plsc.VectorSubcoreMesh(core_axis_name="c", subcore_axis_name="s")


def _deg_kernel(n, rows_per_w, zrows, racc):
    @functools.partial(
        pl.kernel,
        out_type=jax.ShapeDtypeStruct((NC, n, L), jnp.float32),
        mesh=_sc_mesh(),
        scratch_types=[
            pltpu.VMEM((rows_per_w, CHUNK), jnp.int32),
            pltpu.VMEM((CHUNK, L), jnp.float32),
            pltpu.VMEM_SHARED((racc, L), jnp.float32),
        ],
    )
    def deg_kernel(src_hbm, ones_hbm, zer_hbm, deg_out, idx_v, ones_v, acc):
        cid = lax.axis_index("c")
        sid = lax.axis_index("s")
        wid = cid * NS + sid
        pltpu.sync_copy(zer_hbm, acc.at[pl.ds(sid * zrows, zrows), :])
        pltpu.sync_copy(ones_hbm, ones_v)
        pltpu.sync_copy(src_hbm.at[pl.ds(wid * rows_per_w, rows_per_w), :], idx_v)
        plsc.subcore_barrier()

        @pl.loop(0, rows_per_w)
        def _(j):
            pltpu.sync_copy(ones_v, acc.at[idx_v.at[j]], add=True)

        plsc.subcore_barrier()
        rows_out = n // NS
        pltpu.sync_copy(
            acc.at[pl.ds(sid * rows_out, rows_out), :],
            deg_out.at[cid, pl.ds(sid * rows_out, rows_out), :],
        )

    return deg_kernel


def _agg_kernel(n, d, rows_per_w, zrows, racc):
    @functools.partial(
        pl.kernel,
        out_type=jax.ShapeDtypeStruct((NC, n, d), jnp.float32),
        mesh=_sc_mesh(),
        scratch_types=[
            pltpu.VMEM((rows_per_w, CHUNK), jnp.int32),
            pltpu.VMEM((rows_per_w, CHUNK), jnp.int32),
            pltpu.VMEM((CHUNK, d), jnp.float32),
            pltpu.VMEM_SHARED((racc, d), jnp.float32),
        ],
    )
    def agg_kernel(y_hbm, src_hbm, dst_hbm, zer_hbm, s_out, si_v, di_v, rows_v, acc):
        cid = lax.axis_index("c")
        sid = lax.axis_index("s")
        wid = cid * NS + sid
        pltpu.sync_copy(zer_hbm, acc.at[pl.ds(sid * zrows, zrows), :])
        pltpu.sync_copy(src_hbm.at[pl.ds(wid * rows_per_w, rows_per_w), :], si_v)
        pltpu.sync_copy(dst_hbm.at[pl.ds(wid * rows_per_w, rows_per_w), :], di_v)
        plsc.subcore_barrier()

        @pl.loop(0, rows_per_w)
        def _(j):
            pltpu.sync_copy(y_hbm.at[si_v.at[j]], rows_v)
            pltpu.sync_copy(rows_v, acc.at[di_v.at[j]], add=True)

        plsc.subcore_barrier()
        rows_out = n // NS
        pltpu.sync_copy(
            acc.at[pl.ds(sid * rows_out, rows_out), :],
            s_out.at[cid, pl.ds(sid * rows_out, rows_out), :],
        )

    return agg_kernel


def _scale_body(x_ref, d0_ref, d1_ref, y_ref):
    deg = d0_ref[:, 0:1] + d1_ref[:, 0:1]
    r = lax.rsqrt(jnp.maximum(deg, 1.0))
    y_ref[...] = x_ref[...] * r


def _final_body(x_ref, s0_ref, s1_ref, d0_ref, d1_ref, w1_ref, w2_ref, o_ref):
    deg = d0_ref[:, 0:1] + d1_ref[:, 0:1]
    r = lax.rsqrt(jnp.maximum(deg, 1.0))
    t = (s0_ref[...] + s1_ref[...]) * r
    xv = x_ref[...]
    e = jnp.dot(xv + t, w1_ref[...], preferred_element_type=jnp.float32)
    e = e + jnp.dot(t * xv, w2_ref[...], preferred_element_type=jnp.float32)
    o_ref[...] = jnp.where(e >= 0, e, 0.01 * e)


@jax.jit
def kernel(x, edge_index, W1, W2):
    n, d = x.shape
    e = edge_index.shape[1]
    nw = NC * NS
    rows_per_w = pl.cdiv(pl.cdiv(e, CHUNK), nw)
    e_pad = rows_per_w * nw * CHUNK
    zrows = pl.cdiv(n + 1, NS)
    racc = zrows * NS

    src = edge_index[0]
    dst = edge_index[1]
    pad = e_pad - e
    src2d = jnp.concatenate([src, jnp.zeros((pad,), jnp.int32)]).reshape(-1, CHUNK)
    dst2d = jnp.concatenate([dst, jnp.full((pad,), n, jnp.int32)]).reshape(-1, CHUNK)

    ones_c = jnp.ones((CHUNK, L), jnp.float32)
    zer_l = jnp.zeros((zrows, L), jnp.float32)
    zer_d = jnp.zeros((zrows, d), jnp.float32)

    deg2 = _deg_kernel(n, rows_per_w, zrows, racc)(src2d, ones_c, zer_l)
    d0, d1 = deg2[0], deg2[1]

    bn = 1000
    grid = (n // bn,)
    y = pl.pallas_call(
        _scale_body,
        grid=grid,
        in_specs=[
            pl.BlockSpec((bn, d), lambda i: (i, 0)),
            pl.BlockSpec((bn, L), lambda i: (i, 0)),
            pl.BlockSpec((bn, L), lambda i: (i, 0)),
        ],
        out_specs=pl.BlockSpec((bn, d), lambda i: (i, 0)),
        out_shape=jax.ShapeDtypeStruct((n, d), jnp.float32),
    )(x, d0, d1)

    s2 = _agg_kernel(n, d, rows_per_w, zrows, racc)(y, src2d, dst2d, zer_d)

    out = pl.pallas_call(
        _final_body,
        grid=grid,
        in_specs=[
            pl.BlockSpec((bn, d), lambda i: (i, 0)),
            pl.BlockSpec((bn, d), lambda i: (i, 0)),
            pl.BlockSpec((bn, d), lambda i: (i, 0)),
            pl.BlockSpec((bn, L), lambda i: (i, 0)),
            pl.BlockSpec((bn, L), lambda i: (i, 0)),
            pl.BlockSpec((d, d), lambda i: (0, 0)),
            pl.BlockSpec((d, d), lambda i: (0, 0)),
        ],
        out_specs=pl.BlockSpec((bn, d), lambda i: (i, 0)),
        out_shape=jax.ShapeDtypeStruct((n, d), jnp.float32),
    )(x, s2[0], s2[1], d0, d1, W1, W2)
    return out


# speedup vs baseline: 10.7577x; 10.7577x over previous
import functools

import jax
import jax.numpy as jnp
from jax import lax
from jax.experimental import pallas as pl
from jax.experimental.pallas import tpu as pltpu
from jax.experimental.pallas import tpu_sc as plsc

NC = 2
NS = 16
L = 16
CHUNK = 128


def _sc_mesh():
    return plsc.VectorSubcoreMesh(core_axis_name="c", subcore_axis_name="s")


def _deg_kernel(n, d, rows_per_w, zrows, racc):
    @functools.partial(
        pl.kernel,
        out_type=jax.ShapeDtypeStruct((NC, n, d), jnp.float32),
        mesh=_sc_mesh(),
        scratch_types=[
            pltpu.VMEM((rows_per_w, CHUNK), jnp.int32),
            pltpu.VMEM((CHUNK, d), jnp.float32),
            pltpu.VMEM_SHARED((racc, d), jnp.float32),
        ],
    )
    def deg_kernel(src_hbm, ones_hbm, zer_hbm, deg_out, idx_v, ones_v, acc):
        cid = lax.axis_index("c")
        sid = lax.axis_index("s")
        wid = cid * NS + sid
        pltpu.sync_copy(zer_hbm, acc.at[pl.ds(sid * zrows, zrows), :])
        pltpu.sync_copy(ones_hbm, ones_v)
        pltpu.sync_copy(src_hbm.at[pl.ds(wid * rows_per_w, rows_per_w), :], idx_v)
        plsc.subcore_barrier()

        @pl.loop(0, rows_per_w)
        def _(j):
            pltpu.sync_copy(ones_v, acc.at[idx_v.at[j]], add=True)

        plsc.subcore_barrier()
        base = (n // (NS * 8)) * 8
        rem = n - base * NS
        pltpu.sync_copy(
            acc.at[pl.ds(sid * base, base), :],
            deg_out.at[cid, pl.ds(sid * base, base), :],
        )
        if rem:
            @pl.when(sid == 0)
            def _():
                pltpu.sync_copy(
                    acc.at[pl.ds(base * NS, rem), :],
                    deg_out.at[cid, pl.ds(base * NS, rem), :],
                )

    return deg_kernel


def _agg_kernel(n, d, rows_per_w, zrows, racc):
    @functools.partial(
        pl.kernel,
        out_type=jax.ShapeDtypeStruct((NC, n, d), jnp.float32),
        mesh=_sc_mesh(),
        scratch_types=[
            pltpu.VMEM((rows_per_w, CHUNK), jnp.int32),
            pltpu.VMEM((rows_per_w, CHUNK), jnp.int32),
            pltpu.VMEM((CHUNK, d), jnp.float32),
            pltpu.VMEM_SHARED((racc, d), jnp.float32),
        ],
    )
    def agg_kernel(y_hbm, src_hbm, dst_hbm, zer_hbm, s_out, si_v, di_v, rows_v, acc):
        cid = lax.axis_index("c")
        sid = lax.axis_index("s")
        wid = cid * NS + sid
        pltpu.sync_copy(zer_hbm, acc.at[pl.ds(sid * zrows, zrows), :])
        pltpu.sync_copy(src_hbm.at[pl.ds(wid * rows_per_w, rows_per_w), :], si_v)
        pltpu.sync_copy(dst_hbm.at[pl.ds(wid * rows_per_w, rows_per_w), :], di_v)
        plsc.subcore_barrier()

        @pl.loop(0, rows_per_w)
        def _(j):
            pltpu.sync_copy(y_hbm.at[si_v.at[j]], rows_v)
            pltpu.sync_copy(rows_v, acc.at[di_v.at[j]], add=True)

        plsc.subcore_barrier()
        base = (n // (NS * 8)) * 8
        rem = n - base * NS
        pltpu.sync_copy(
            acc.at[pl.ds(sid * base, base), :],
            s_out.at[cid, pl.ds(sid * base, base), :],
        )
        if rem:
            @pl.when(sid == 0)
            def _():
                pltpu.sync_copy(
                    acc.at[pl.ds(base * NS, rem), :],
                    s_out.at[cid, pl.ds(base * NS, rem), :],
                )

    return agg_kernel


def _scale_body(x_ref, d0_ref, d1_ref, y_ref):
    deg = d0_ref[:, 0:1] + d1_ref[:, 0:1]
    r = lax.rsqrt(jnp.maximum(deg, 1.0))
    y_ref[...] = x_ref[...] * r


def _final_body(x_ref, s0_ref, s1_ref, d0_ref, d1_ref, w1_ref, w2_ref, o_ref):
    deg = d0_ref[:, 0:1] + d1_ref[:, 0:1]
    r = lax.rsqrt(jnp.maximum(deg, 1.0))
    t = (s0_ref[...] + s1_ref[...]) * r
    xv = x_ref[...]
    e = jnp.dot(xv + t, w1_ref[...], preferred_element_type=jnp.float32)
    e = e + jnp.dot(t * xv, w2_ref[...], preferred_element_type=jnp.float32)
    o_ref[...] = jnp.where(e >= 0, e, 0.01 * e)


@jax.jit
def kernel(x, edge_index, W1, W2):
    n, d = x.shape
    e = edge_index.shape[1]
    nw = NC * NS
    rows_per_w = 8 * pl.cdiv(pl.cdiv(e, CHUNK), nw * 8)
    e_pad = rows_per_w * nw * CHUNK
    zrows = 8 * pl.cdiv(n + 1, NS * 8)
    racc = zrows * NS

    src = edge_index[0]
    dst = edge_index[1]
    pad = e_pad - e
    src2d = jnp.concatenate([src, jnp.full((pad,), n, jnp.int32)]).reshape(-1, CHUNK)
    dst2d = jnp.concatenate([dst, jnp.full((pad,), n, jnp.int32)]).reshape(-1, CHUNK)

    ones_c = jnp.ones((CHUNK, d), jnp.float32)
    zer_d = jnp.zeros((zrows, d), jnp.float32)

    deg2 = _deg_kernel(n, d, rows_per_w, zrows, racc)(src2d, ones_c, zer_d)
    d0, d1 = deg2[0], deg2[1]

    bn = 1000
    grid = (n // bn,)
    y = pl.pallas_call(
        _scale_body,
        grid=grid,
        in_specs=[
            pl.BlockSpec((bn, d), lambda i: (i, 0)),
            pl.BlockSpec((bn, d), lambda i: (i, 0)),
            pl.BlockSpec((bn, d), lambda i: (i, 0)),
        ],
        out_specs=pl.BlockSpec((bn, d), lambda i: (i, 0)),
        out_shape=jax.ShapeDtypeStruct((n, d), jnp.float32),
    )(x, d0, d1)

    y_g = jnp.concatenate([y, jnp.zeros((8, d), jnp.float32)], axis=0)
    s2 = _agg_kernel(n, d, rows_per_w, zrows, racc)(y_g, src2d, dst2d, zer_d)

    out = pl.pallas_call(
        _final_body,
        grid=grid,
        in_specs=[
            pl.BlockSpec((bn, d), lambda i: (i, 0)),
            pl.BlockSpec((bn, d), lambda i: (i, 0)),
            pl.BlockSpec((bn, d), lambda i: (i, 0)),
            pl.BlockSpec((bn, d), lambda i: (i, 0)),
            pl.BlockSpec((bn, d), lambda i: (i, 0)),
            pl.BlockSpec((d, d), lambda i: (0, 0)),
            pl.BlockSpec((d, d), lambda i: (0, 0)),
        ],
        out_specs=pl.BlockSpec((bn, d), lambda i: (i, 0)),
        out_shape=jax.ShapeDtypeStruct((n, d), jnp.float32),
    )(x, s2[0], s2[1], d0, d1, W1, W2)
    return out


# speedup vs baseline: 21.8657x; 2.0326x over previous
import functools

import jax
import jax.numpy as jnp
from jax import lax
from jax.experimental import pallas as pl
from jax.experimental.pallas import tpu as pltpu
from jax.experimental.pallas import tpu_sc as plsc

NC = 2
NS = 16
L = 16
CHUNK = 128


def _sc_mesh():
    return plsc.VectorSubcoreMesh(core_axis_name="c", subcore_axis_name="s")


def _deg_kernel(n, d, rows_per_w, zrows, racc):
    @functools.partial(
        pl.kernel,
        out_type=jax.ShapeDtypeStruct((NC, n, d), jnp.float32),
        mesh=_sc_mesh(),
        scratch_types=[
            pltpu.VMEM((rows_per_w, CHUNK), jnp.int32),
            pltpu.VMEM((CHUNK, d), jnp.float32),
            pltpu.VMEM_SHARED((racc, d), jnp.float32),
        ],
    )
    def deg_kernel(src_hbm, ones_hbm, zer_hbm, deg_out, idx_v, ones_v, acc):
        cid = lax.axis_index("c")
        sid = lax.axis_index("s")
        wid = cid * NS + sid
        pltpu.sync_copy(zer_hbm, acc.at[pl.ds(sid * zrows, zrows), :])
        pltpu.sync_copy(ones_hbm, ones_v)
        pltpu.sync_copy(src_hbm.at[pl.ds(wid * rows_per_w, rows_per_w), :], idx_v)
        plsc.subcore_barrier()

        @pl.loop(0, rows_per_w)
        def _(j):
            pltpu.sync_copy(ones_v, acc.at[idx_v.at[j]], add=True)

        plsc.subcore_barrier()
        base = (n // (NS * 8)) * 8
        rem = n - base * NS
        pltpu.sync_copy(
            acc.at[pl.ds(sid * base, base), :],
            deg_out.at[cid, pl.ds(sid * base, base), :],
        )
        if rem:
            @pl.when(sid == 0)
            def _():
                pltpu.sync_copy(
                    acc.at[pl.ds(base * NS, rem), :],
                    deg_out.at[cid, pl.ds(base * NS, rem), :],
                )

    return deg_kernel


def _agg_kernel(n, d, rows_per_w, zrows, racc):
    @functools.partial(
        pl.kernel,
        out_type=jax.ShapeDtypeStruct((NC, n, d), jnp.float32),
        mesh=_sc_mesh(),
        scratch_types=[
            pltpu.VMEM((rows_per_w, CHUNK), jnp.int32),
            pltpu.VMEM((rows_per_w, CHUNK), jnp.int32),
            pltpu.VMEM((CHUNK, d), jnp.float32),
            pltpu.VMEM_SHARED((racc, d), jnp.float32),
        ],
    )
    def agg_kernel(y_hbm, src_hbm, dst_hbm, zer_hbm, s_out, si_v, di_v, rows_v, acc):
        cid = lax.axis_index("c")
        sid = lax.axis_index("s")
        wid = cid * NS + sid
        pltpu.sync_copy(zer_hbm, acc.at[pl.ds(sid * zrows, zrows), :])
        pltpu.sync_copy(src_hbm.at[pl.ds(wid * rows_per_w, rows_per_w), :], si_v)
        pltpu.sync_copy(dst_hbm.at[pl.ds(wid * rows_per_w, rows_per_w), :], di_v)
        plsc.subcore_barrier()

        @pl.loop(0, rows_per_w)
        def _(j):
            pltpu.sync_copy(y_hbm.at[si_v.at[j]], rows_v)
            pltpu.sync_copy(rows_v, acc.at[di_v.at[j]], add=True)

        plsc.subcore_barrier()
        base = (n // (NS * 8)) * 8
        rem = n - base * NS
        pltpu.sync_copy(
            acc.at[pl.ds(sid * base, base), :],
            s_out.at[cid, pl.ds(sid * base, base), :],
        )
        if rem:
            @pl.when(sid == 0)
            def _():
                pltpu.sync_copy(
                    acc.at[pl.ds(base * NS, rem), :],
                    s_out.at[cid, pl.ds(base * NS, rem), :],
                )

    return agg_kernel


def _scale_body(x_ref, d0_ref, d1_ref, y_ref):
    deg = d0_ref[:, 0:1] + d1_ref[:, 0:1]
    r = lax.rsqrt(jnp.maximum(deg, 1.0))
    y_ref[...] = x_ref[...] * r


def _final_body(x_ref, s0_ref, s1_ref, d0_ref, d1_ref, w1_ref, w2_ref, o_ref):
    deg = d0_ref[:, 0:1] + d1_ref[:, 0:1]
    r = lax.rsqrt(jnp.maximum(deg, 1.0))
    t = (s0_ref[...] + s1_ref[...]) * r
    xv = x_ref[...]
    e = jnp.dot(xv + t, w1_ref[...], preferred_element_type=jnp.float32)
    e = e + jnp.dot(t * xv, w2_ref[...], preferred_element_type=jnp.float32)
    o_ref[...] = jnp.where(e >= 0, e, 0.01 * e)


@jax.jit
def kernel(x, edge_index, W1, W2):
    n, d = x.shape
    e = edge_index.shape[1]
    nw = NC * NS
    rows_per_w = 8 * pl.cdiv(pl.cdiv(e, CHUNK), nw * 8)
    e_pad = rows_per_w * nw * CHUNK
    zrows = 8 * pl.cdiv(n + 512, NS * 8)
    racc = zrows * NS
    sinks = racc - n

    src = edge_index[0]
    dst = edge_index[1]
    pad = e_pad - e
    cyc = jnp.arange(pad, dtype=jnp.int32)
    src_deg = jnp.concatenate([src, n + cyc % sinks]).reshape(-1, CHUNK)
    src_agg = jnp.concatenate([src, n + cyc % 8]).reshape(-1, CHUNK)
    dst2d = jnp.concatenate([dst, n + cyc % sinks]).reshape(-1, CHUNK)

    ones_c = jnp.ones((CHUNK, d), jnp.float32)
    zer_d = jnp.zeros((zrows, d), jnp.float32)

    deg2 = _deg_kernel(n, d, rows_per_w, zrows, racc)(src_deg, ones_c, zer_d)
    d0, d1 = deg2[0], deg2[1]

    bn = 1000
    grid = (n // bn,)
    y = pl.pallas_call(
        _scale_body,
        grid=grid,
        in_specs=[
            pl.BlockSpec((bn, d), lambda i: (i, 0)),
            pl.BlockSpec((bn, d), lambda i: (i, 0)),
            pl.BlockSpec((bn, d), lambda i: (i, 0)),
        ],
        out_specs=pl.BlockSpec((bn, d), lambda i: (i, 0)),
        out_shape=jax.ShapeDtypeStruct((n, d), jnp.float32),
    )(x, d0, d1)

    y_g = jnp.concatenate([y, jnp.zeros((8, d), jnp.float32)], axis=0)
    s2 = _agg_kernel(n, d, rows_per_w, zrows, racc)(y_g, src_agg, dst2d, zer_d)

    out = pl.pallas_call(
        _final_body,
        grid=grid,
        in_specs=[
            pl.BlockSpec((bn, d), lambda i: (i, 0)),
            pl.BlockSpec((bn, d), lambda i: (i, 0)),
            pl.BlockSpec((bn, d), lambda i: (i, 0)),
            pl.BlockSpec((bn, d), lambda i: (i, 0)),
            pl.BlockSpec((bn, d), lambda i: (i, 0)),
            pl.BlockSpec((d, d), lambda i: (0, 0)),
            pl.BlockSpec((d, d), lambda i: (0, 0)),
        ],
        out_specs=pl.BlockSpec((bn, d), lambda i: (i, 0)),
        out_shape=jax.ShapeDtypeStruct((n, d), jnp.float32),
    )(x, s2[0], s2[1], d0, d1, W1, W2)
    return out


# speedup vs baseline: 26.8702x; 1.2289x over previous
import functools

import jax
import jax.numpy as jnp
from jax import lax
from jax.experimental import pallas as pl
from jax.experimental.pallas import tpu as pltpu
from jax.experimental.pallas import tpu_sc as plsc

NC = 2
NS = 16
L = 16
CHUNK = 128
ACHUNK = 64


def _sc_mesh():
    return plsc.VectorSubcoreMesh(core_axis_name="c", subcore_axis_name="s")


def _deg_kernel(n, d, rows_per_w, zrows, racc):
    @functools.partial(
        pl.kernel,
        out_type=jax.ShapeDtypeStruct((NC, n, d), jnp.float32),
        mesh=_sc_mesh(),
        scratch_types=[
            pltpu.VMEM((rows_per_w, CHUNK), jnp.int32),
            pltpu.VMEM((CHUNK, d), jnp.float32),
            pltpu.VMEM_SHARED((racc, d), jnp.float32),
            pltpu.SemaphoreType.DMA,
            pltpu.SemaphoreType.DMA,
        ],
    )
    def deg_kernel(src_hbm, ones_hbm, zer_hbm, deg_out, idx_v, ones_v, acc,
                   sem_a, sem_b):
        cid = lax.axis_index("c")
        sid = lax.axis_index("s")
        wid = cid * NS + sid
        pltpu.sync_copy(zer_hbm, acc.at[pl.ds(sid * zrows, zrows), :])
        pltpu.sync_copy(ones_hbm, ones_v)
        pltpu.sync_copy(src_hbm.at[pl.ds(wid * rows_per_w, rows_per_w), :], idx_v)
        plsc.subcore_barrier()

        pltpu.async_copy(ones_v, acc.at[idx_v.at[0]], sem_a, add=True)

        @pl.loop(0, rows_per_w // 2 - 1)
        def _(g):
            j = 2 * g
            pltpu.async_copy(ones_v, acc.at[idx_v.at[j + 1]], sem_b, add=True)
            pltpu.make_async_copy(ones_v, acc.at[idx_v.at[j]], sem_a).wait()
            pltpu.async_copy(ones_v, acc.at[idx_v.at[j + 2]], sem_a, add=True)
            pltpu.make_async_copy(ones_v, acc.at[idx_v.at[j + 1]], sem_b).wait()

        last = rows_per_w - 1
        pltpu.async_copy(ones_v, acc.at[idx_v.at[last]], sem_b, add=True)
        pltpu.make_async_copy(ones_v, acc.at[idx_v.at[last - 1]], sem_a).wait()
        pltpu.make_async_copy(ones_v, acc.at[idx_v.at[last]], sem_b).wait()

        plsc.subcore_barrier()
        base = (n // (NS * 8)) * 8
        rem = n - base * NS
        pltpu.sync_copy(
            acc.at[pl.ds(sid * base, base), :],
            deg_out.at[cid, pl.ds(sid * base, base), :],
        )
        if rem:
            @pl.when(sid == 0)
            def _():
                pltpu.sync_copy(
                    acc.at[pl.ds(base * NS, rem), :],
                    deg_out.at[cid, pl.ds(base * NS, rem), :],
                )

    return deg_kernel


def _agg_kernel(n, d, rows_per_w, zrows, racc):
    hrows = rows_per_w // 2

    @functools.partial(
        pl.kernel,
        out_type=jax.ShapeDtypeStruct((NC, n, d), jnp.float32),
        mesh=_sc_mesh(),
        scratch_types=[
            pltpu.VMEM((hrows, ACHUNK), jnp.int32),
            pltpu.VMEM((hrows, ACHUNK), jnp.int32),
            pltpu.VMEM((ACHUNK, d), jnp.float32),
            pltpu.VMEM((ACHUNK, d), jnp.float32),
            pltpu.VMEM_SHARED((racc, d), jnp.float32),
            pltpu.SemaphoreType.DMA,
            pltpu.SemaphoreType.DMA,
        ],
    )
    def agg_kernel(y_hbm, src_hbm, dst_hbm, zer_hbm, s_out, si_v, di_v,
                   rows_a, rows_b, acc, sem_a, sem_b):
        cid = lax.axis_index("c")
        sid = lax.axis_index("s")
        wid = cid * NS + sid
        pltpu.sync_copy(zer_hbm, acc.at[pl.ds(sid * zrows, zrows), :])
        plsc.subcore_barrier()

        for h in range(2):
            row0 = wid * rows_per_w + h * hrows
            pltpu.sync_copy(src_hbm.at[pl.ds(row0, hrows), :], si_v)
            pltpu.sync_copy(dst_hbm.at[pl.ds(row0, hrows), :], di_v)

            pltpu.async_copy(y_hbm.at[si_v.at[0]], rows_a, sem_a)

            @pl.loop(0, hrows // 2)
            def _(g):
                j = 2 * g
                pltpu.async_copy(y_hbm.at[si_v.at[j + 1]], rows_b, sem_b)
                pltpu.make_async_copy(y_hbm.at[si_v.at[j]], rows_a, sem_a).wait()
                pltpu.sync_copy(rows_a, acc.at[di_v.at[j]], add=True)
                jn = jnp.minimum(j + 2, hrows - 1)
                pltpu.async_copy(y_hbm.at[si_v.at[jn]], rows_a, sem_a)
                pltpu.make_async_copy(y_hbm.at[si_v.at[j + 1]], rows_b, sem_b).wait()
                pltpu.sync_copy(rows_b, acc.at[di_v.at[j + 1]], add=True)

            pltpu.make_async_copy(y_hbm.at[si_v.at[0]], rows_a, sem_a).wait()

        plsc.subcore_barrier()
        base = (n // (NS * 8)) * 8
        rem = n - base * NS
        pltpu.sync_copy(
            acc.at[pl.ds(sid * base, base), :],
            s_out.at[cid, pl.ds(sid * base, base), :],
        )
        if rem:
            @pl.when(sid == 0)
            def _():
                pltpu.sync_copy(
                    acc.at[pl.ds(base * NS, rem), :],
                    s_out.at[cid, pl.ds(base * NS, rem), :],
                )

    return agg_kernel


def _scale_body(x_ref, d0_ref, d1_ref, y_ref):
    deg = d0_ref[:, 0:1] + d1_ref[:, 0:1]
    r = lax.rsqrt(jnp.maximum(deg, 1.0))
    y_ref[...] = x_ref[...] * r


def _final_body(x_ref, s0_ref, s1_ref, d0_ref, d1_ref, w1_ref, w2_ref, o_ref):
    deg = d0_ref[:, 0:1] + d1_ref[:, 0:1]
    r = lax.rsqrt(jnp.maximum(deg, 1.0))
    t = (s0_ref[...] + s1_ref[...]) * r
    xv = x_ref[...]
    e = jnp.dot(xv + t, w1_ref[...], preferred_element_type=jnp.float32)
    e = e + jnp.dot(t * xv, w2_ref[...], preferred_element_type=jnp.float32)
    o_ref[...] = jnp.where(e >= 0, e, 0.01 * e)


@jax.jit
def kernel(x, edge_index, W1, W2):
    n, d = x.shape
    e = edge_index.shape[1]
    nw = NC * NS
    rows_deg = 8 * pl.cdiv(pl.cdiv(e, CHUNK), nw * 8)
    rows_agg = 8 * pl.cdiv(pl.cdiv(e, ACHUNK), nw * 8)
    pad_deg = rows_deg * nw * CHUNK - e
    pad_agg = rows_agg * nw * ACHUNK - e
    zrows = 8 * pl.cdiv(n + 1, NS * 8)
    racc = zrows * NS
    sinks = racc - n

    src = edge_index[0]
    dst = edge_index[1]
    cyc_d = jnp.arange(pad_deg, dtype=jnp.int32)
    cyc_a = jnp.arange(pad_agg, dtype=jnp.int32)
    src_deg = jnp.concatenate([src, n + cyc_d % sinks]).reshape(-1, CHUNK)
    src_agg = jnp.concatenate([src, n + cyc_a % 8]).reshape(-1, ACHUNK)
    dst2d = jnp.concatenate([dst, n + cyc_a % sinks]).reshape(-1, ACHUNK)

    ones_c = jnp.ones((CHUNK, d), jnp.float32)
    zer_d = jnp.zeros((zrows, d), jnp.float32)

    deg2 = _deg_kernel(n, d, rows_deg, zrows, racc)(src_deg, ones_c, zer_d)
    d0, d1 = deg2[0], deg2[1]

    bn = 1000
    grid = (n // bn,)
    y = pl.pallas_call(
        _scale_body,
        grid=grid,
        in_specs=[
            pl.BlockSpec((bn, d), lambda i: (i, 0)),
            pl.BlockSpec((bn, d), lambda i: (i, 0)),
            pl.BlockSpec((bn, d), lambda i: (i, 0)),
        ],
        out_specs=pl.BlockSpec((bn, d), lambda i: (i, 0)),
        out_shape=jax.ShapeDtypeStruct((n, d), jnp.float32),
    )(x, d0, d1)

    y_g = jnp.concatenate([y, jnp.zeros((8, d), jnp.float32)], axis=0)
    s2 = _agg_kernel(n, d, rows_agg, zrows, racc)(y_g, src_agg, dst2d, zer_d)

    out = pl.pallas_call(
        _final_body,
        grid=grid,
        in_specs=[
            pl.BlockSpec((bn, d), lambda i: (i, 0)),
            pl.BlockSpec((bn, d), lambda i: (i, 0)),
            pl.BlockSpec((bn, d), lambda i: (i, 0)),
            pl.BlockSpec((bn, d), lambda i: (i, 0)),
            pl.BlockSpec((bn, d), lambda i: (i, 0)),
            pl.BlockSpec((d, d), lambda i: (0, 0)),
            pl.BlockSpec((d, d), lambda i: (0, 0)),
        ],
        out_specs=pl.BlockSpec((bn, d), lambda i: (i, 0)),
        out_shape=jax.ShapeDtypeStruct((n, d), jnp.float32),
    )(x, s2[0], s2[1], d0, d1, W1, W2)
    return out


# speedup vs baseline: 33.3875x; 1.2425x over previous
import dataclasses
import functools

import jax
import jax.numpy as jnp
from jax import lax
from jax.experimental import pallas as pl
from jax.experimental.pallas import tpu as pltpu
from jax.experimental.pallas import tpu_sc as plsc

NC = 2
NS = 16
L = 16
CHUNK = 128
ACHUNK = 64


def _sc_mesh():
    return plsc.VectorSubcoreMesh(core_axis_name="c", subcore_axis_name="s")


def _sc_params():
    cp = pltpu.CompilerParams()
    if "needs_layout_passes" in pltpu.CompilerParams.__dataclass_fields__:
        cp = dataclasses.replace(cp, needs_layout_passes=False)
    return cp


def _deg_kernel(n, epw, brows):
    @functools.partial(
        pl.kernel,
        out_type=jax.ShapeDtypeStruct((NC, brows, 128), jnp.float32),
        mesh=_sc_mesh(),
        scratch_types=[
            pltpu.VMEM((epw,), jnp.int32),
            pltpu.VMEM((brows, 128), jnp.float32),
            pltpu.VMEM((brows,), jnp.int32),
            pltpu.VMEM_SHARED((brows, 128), jnp.float32),
        ],
        compiler_params=_sc_params(),
    )
    def deg_kernel(src_hbm, zer_hbm, iota_hbm, deg_out, idx_v, hist_v, iota_v,
                   acc):
        cid = lax.axis_index("c")
        sid = lax.axis_index("s")
        wid = cid * NS + sid

        @pl.when(sid < brows // 8)
        def _():
            pltpu.sync_copy(zer_hbm.at[pl.ds(0, 8), :],
                            acc.at[pl.ds(sid * 8, 8), :])

        pltpu.sync_copy(zer_hbm, hist_v)
        pltpu.sync_copy(iota_hbm, iota_v)
        pltpu.sync_copy(src_hbm.at[pl.ds(wid * epw, epw)], idx_v)
        ones16 = jnp.ones((16,), jnp.float32)

        @pl.loop(0, epw, step=16)
        def _(i):
            ix = idx_v[pl.ds(i, 16)]
            r = lax.shift_right_logical(ix, 7)
            c = lax.bitwise_and(ix, 127)
            plsc.addupdate_scatter(hist_v, [r, c], ones16)

        plsc.subcore_barrier()
        pltpu.sync_copy(hist_v, acc.at[iota_v], add=True)
        plsc.subcore_barrier()

        @pl.when(sid == 0)
        def _():
            pltpu.sync_copy(acc, deg_out.at[cid])

    return deg_kernel


def _agg_kernel(n, d, rows_per_w, zrows, racc):
    hrows = rows_per_w // 2

    @functools.partial(
        pl.kernel,
        out_type=jax.ShapeDtypeStruct((NC, n, d), jnp.float32),
        mesh=_sc_mesh(),
        scratch_types=[
            pltpu.VMEM((hrows, ACHUNK), jnp.int32),
            pltpu.VMEM((hrows, ACHUNK), jnp.int32),
            pltpu.VMEM((ACHUNK, d), jnp.float32),
            pltpu.VMEM((ACHUNK, d), jnp.float32),
            pltpu.VMEM_SHARED((racc, d), jnp.float32),
            pltpu.SemaphoreType.DMA,
            pltpu.SemaphoreType.DMA,
        ],
    )
    def agg_kernel(y_hbm, src_hbm, dst_hbm, zer_hbm, s_out, si_v, di_v,
                   rows_a, rows_b, acc, sem_a, sem_b):
        cid = lax.axis_index("c")
        sid = lax.axis_index("s")
        wid = cid * NS + sid
        pltpu.sync_copy(zer_hbm, acc.at[pl.ds(sid * zrows, zrows), :])
        plsc.subcore_barrier()

        for h in range(2):
            row0 = wid * rows_per_w + h * hrows
            pltpu.sync_copy(src_hbm.at[pl.ds(row0, hrows), :], si_v)
            pltpu.sync_copy(dst_hbm.at[pl.ds(row0, hrows), :], di_v)

            pltpu.async_copy(y_hbm.at[si_v.at[0]], rows_a, sem_a)

            @pl.loop(0, hrows // 2)
            def _(g):
                j = 2 * g
                pltpu.async_copy(y_hbm.at[si_v.at[j + 1]], rows_b, sem_b)
                pltpu.make_async_copy(y_hbm.at[si_v.at[j]], rows_a, sem_a).wait()
                pltpu.sync_copy(rows_a, acc.at[di_v.at[j]], add=True)
                jn = jnp.minimum(j + 2, hrows - 1)
                pltpu.async_copy(y_hbm.at[si_v.at[jn]], rows_a, sem_a)
                pltpu.make_async_copy(y_hbm.at[si_v.at[j + 1]], rows_b, sem_b).wait()
                pltpu.sync_copy(rows_b, acc.at[di_v.at[j + 1]], add=True)

            pltpu.make_async_copy(y_hbm.at[si_v.at[0]], rows_a, sem_a).wait()

        plsc.subcore_barrier()
        base = (n // (NS * 8)) * 8
        rem = n - base * NS
        pltpu.sync_copy(
            acc.at[pl.ds(sid * base, base), :],
            s_out.at[cid, pl.ds(sid * base, base), :],
        )
        if rem:
            @pl.when(sid == 0)
            def _():
                pltpu.sync_copy(
                    acc.at[pl.ds(base * NS, rem), :],
                    s_out.at[cid, pl.ds(base * NS, rem), :],
                )

    return agg_kernel


def _scale_body(x_ref, d0_ref, d1_ref, y_ref):
    deg = d0_ref[...] + d1_ref[...]
    r = lax.rsqrt(jnp.maximum(deg, 1.0))
    y_ref[...] = x_ref[...] * r


def _final_body(x_ref, s0_ref, s1_ref, d0_ref, d1_ref, w1_ref, w2_ref, o_ref):
    deg = d0_ref[...] + d1_ref[...]
    r = lax.rsqrt(jnp.maximum(deg, 1.0))
    t = (s0_ref[...] + s1_ref[...]) * r
    xv = x_ref[...]
    e = jnp.dot(xv + t, w1_ref[...], preferred_element_type=jnp.float32)
    e = e + jnp.dot(t * xv, w2_ref[...], preferred_element_type=jnp.float32)
    o_ref[...] = jnp.where(e >= 0, e, 0.01 * e)


@jax.jit
def kernel(x, edge_index, W1, W2):
    n, d = x.shape
    e = edge_index.shape[1]
    nw = NC * NS
    epw = 8 * pl.cdiv(e, nw * 8)
    brows = 8 * pl.cdiv(n + 1, 128 * 8)
    rows_agg = 8 * pl.cdiv(pl.cdiv(e, ACHUNK), nw * 8)
    pad_deg = epw * nw - e
    pad_agg = rows_agg * nw * ACHUNK - e
    zrows = 8 * pl.cdiv(n + 1, NS * 8)
    racc = zrows * NS
    sinks = racc - n

    src = edge_index[0]
    dst = edge_index[1]
    cyc_d = jnp.arange(pad_deg, dtype=jnp.int32)
    cyc_a = jnp.arange(pad_agg, dtype=jnp.int32)
    src_deg = jnp.concatenate([src, n + cyc_d % (brows * 128 - n)])
    src_agg = jnp.concatenate([src, n + cyc_a % 8]).reshape(-1, ACHUNK)
    dst2d = jnp.concatenate([dst, n + cyc_a % sinks]).reshape(-1, ACHUNK)

    zer_d = jnp.zeros((zrows, d), jnp.float32)
    zer_b = jnp.zeros((brows, 128), jnp.float32)
    iota_b = jnp.arange(brows, dtype=jnp.int32)

    deg2 = _deg_kernel(n, epw, brows)(src_deg, zer_b, iota_b)
    degf = deg2.reshape(NC, brows * 128)
    d0 = degf[0, :n, None]
    d1 = degf[1, :n, None]

    bn = 1000
    grid = (n // bn,)
    y = pl.pallas_call(
        _scale_body,
        grid=grid,
        in_specs=[
            pl.BlockSpec((bn, d), lambda i: (i, 0)),
            pl.BlockSpec((bn, 1), lambda i: (i, 0)),
            pl.BlockSpec((bn, 1), lambda i: (i, 0)),
        ],
        out_specs=pl.BlockSpec((bn, d), lambda i: (i, 0)),
        out_shape=jax.ShapeDtypeStruct((n, d), jnp.float32),
    )(x, d0, d1)

    y_g = jnp.concatenate([y, jnp.zeros((8, d), jnp.float32)], axis=0)
    s2 = _agg_kernel(n, d, rows_agg, zrows, racc)(y_g, src_agg, dst2d, zer_d)

    out = pl.pallas_call(
        _final_body,
        grid=grid,
        in_specs=[
            pl.BlockSpec((bn, d), lambda i: (i, 0)),
            pl.BlockSpec((bn, d), lambda i: (i, 0)),
            pl.BlockSpec((bn, d), lambda i: (i, 0)),
            pl.BlockSpec((bn, 1), lambda i: (i, 0)),
            pl.BlockSpec((bn, 1), lambda i: (i, 0)),
            pl.BlockSpec((d, d), lambda i: (0, 0)),
            pl.BlockSpec((d, d), lambda i: (0, 0)),
        ],
        out_specs=pl.BlockSpec((bn, d), lambda i: (i, 0)),
        out_shape=jax.ShapeDtypeStruct((n, d), jnp.float32),
    )(x, s2[0], s2[1], d0, d1, W1, W2)
    return out


# speedup vs baseline: 33.4853x; 1.0029x over previous
import dataclasses
import functools

import jax
import jax.numpy as jnp
from jax import lax
from jax.experimental import pallas as pl
from jax.experimental.pallas import tpu as pltpu
from jax.experimental.pallas import tpu_sc as plsc

NC = 2
NS = 16
L = 16
CHUNK = 128
ACHUNK = 128
ASLAB = 16


def _sc_mesh():
    return plsc.VectorSubcoreMesh(core_axis_name="c", subcore_axis_name="s")


def _sc_params():
    cp = pltpu.CompilerParams()
    if "needs_layout_passes" in pltpu.CompilerParams.__dataclass_fields__:
        cp = dataclasses.replace(cp, needs_layout_passes=False)
    return cp


def _deg_kernel(n, epw, brows):
    @functools.partial(
        pl.kernel,
        out_type=jax.ShapeDtypeStruct((NC, brows, 128), jnp.float32),
        mesh=_sc_mesh(),
        scratch_types=[
            pltpu.VMEM((epw,), jnp.int32),
            pltpu.VMEM((brows, 128), jnp.float32),
            pltpu.VMEM((brows,), jnp.int32),
            pltpu.VMEM_SHARED((brows, 128), jnp.float32),
        ],
        compiler_params=_sc_params(),
    )
    def deg_kernel(src_hbm, zer_hbm, iota_hbm, deg_out, idx_v, hist_v, iota_v,
                   acc):
        cid = lax.axis_index("c")
        sid = lax.axis_index("s")
        wid = cid * NS + sid

        @pl.when(sid < brows // 8)
        def _():
            pltpu.sync_copy(zer_hbm.at[pl.ds(0, 8), :],
                            acc.at[pl.ds(sid * 8, 8), :])

        pltpu.sync_copy(zer_hbm, hist_v)
        pltpu.sync_copy(iota_hbm, iota_v)
        pltpu.sync_copy(src_hbm.at[pl.ds(wid * epw, epw)], idx_v)
        ones16 = jnp.ones((16,), jnp.float32)

        @pl.loop(0, epw, step=16)
        def _(i):
            ix = idx_v[pl.ds(i, 16)]
            r = lax.shift_right_logical(ix, 7)
            c = lax.bitwise_and(ix, 127)
            plsc.addupdate_scatter(hist_v, [r, c], ones16)

        plsc.subcore_barrier()
        pltpu.sync_copy(hist_v, acc.at[iota_v], add=True)
        plsc.subcore_barrier()

        @pl.when(sid == 0)
        def _():
            pltpu.sync_copy(acc, deg_out.at[cid])

    return deg_kernel


def _agg_kernel(n, d, rows_per_w, zrows, racc):
    hrows = ASLAB
    nslab = rows_per_w // hrows

    @functools.partial(
        pl.kernel,
        out_type=jax.ShapeDtypeStruct((NC, n, d), jnp.float32),
        mesh=_sc_mesh(),
        scratch_types=[
            pltpu.VMEM((hrows, ACHUNK), jnp.int32),
            pltpu.VMEM((hrows, ACHUNK), jnp.int32),
            pltpu.VMEM((ACHUNK, d), jnp.float32),
            pltpu.VMEM((ACHUNK, d), jnp.float32),
            pltpu.VMEM_SHARED((racc, d), jnp.float32),
            pltpu.SemaphoreType.DMA,
            pltpu.SemaphoreType.DMA,
        ],
    )
    def agg_kernel(y_hbm, src_hbm, dst_hbm, zer_hbm, s_out, si_v, di_v,
                   rows_a, rows_b, acc, sem_a, sem_b):
        cid = lax.axis_index("c")
        sid = lax.axis_index("s")
        wid = cid * NS + sid
        pltpu.sync_copy(zer_hbm, acc.at[pl.ds(sid * zrows, zrows), :])
        plsc.subcore_barrier()

        for h in range(nslab):
            row0 = wid * rows_per_w + h * hrows
            pltpu.sync_copy(src_hbm.at[pl.ds(row0, hrows), :], si_v)
            pltpu.sync_copy(dst_hbm.at[pl.ds(row0, hrows), :], di_v)

            pltpu.async_copy(y_hbm.at[si_v.at[0]], rows_a, sem_a)

            @pl.loop(0, hrows // 2)
            def _(g):
                j = 2 * g
                pltpu.async_copy(y_hbm.at[si_v.at[j + 1]], rows_b, sem_b)
                pltpu.make_async_copy(y_hbm.at[si_v.at[j]], rows_a, sem_a).wait()
                pltpu.sync_copy(rows_a, acc.at[di_v.at[j]], add=True)
                jn = jnp.minimum(j + 2, hrows - 1)
                pltpu.async_copy(y_hbm.at[si_v.at[jn]], rows_a, sem_a)
                pltpu.make_async_copy(y_hbm.at[si_v.at[j + 1]], rows_b, sem_b).wait()
                pltpu.sync_copy(rows_b, acc.at[di_v.at[j + 1]], add=True)

            pltpu.make_async_copy(y_hbm.at[si_v.at[0]], rows_a, sem_a).wait()

        plsc.subcore_barrier()
        base = (n // (NS * 8)) * 8
        rem = n - base * NS
        pltpu.sync_copy(
            acc.at[pl.ds(sid * base, base), :],
            s_out.at[cid, pl.ds(sid * base, base), :],
        )
        if rem:
            @pl.when(sid == 0)
            def _():
                pltpu.sync_copy(
                    acc.at[pl.ds(base * NS, rem), :],
                    s_out.at[cid, pl.ds(base * NS, rem), :],
                )

    return agg_kernel


def _scale_body(x_ref, d0_ref, d1_ref, y_ref):
    deg = d0_ref[...] + d1_ref[...]
    r = lax.rsqrt(jnp.maximum(deg, 1.0))
    y_ref[...] = x_ref[...] * r


def _final_body(x_ref, s0_ref, s1_ref, d0_ref, d1_ref, w1_ref, w2_ref, o_ref):
    deg = d0_ref[...] + d1_ref[...]
    r = lax.rsqrt(jnp.maximum(deg, 1.0))
    t = (s0_ref[...] + s1_ref[...]) * r
    xv = x_ref[...]
    e = jnp.dot(xv + t, w1_ref[...], preferred_element_type=jnp.float32)
    e = e + jnp.dot(t * xv, w2_ref[...], preferred_element_type=jnp.float32)
    o_ref[...] = jnp.where(e >= 0, e, 0.01 * e)


@jax.jit
def kernel(x, edge_index, W1, W2):
    n, d = x.shape
    e = edge_index.shape[1]
    nw = NC * NS
    epw = 8 * pl.cdiv(e, nw * 8)
    brows = 8 * pl.cdiv(n + 1, 128 * 8)
    rows_agg = 8 * pl.cdiv(pl.cdiv(e, ACHUNK), nw * 8)
    pad_deg = epw * nw - e
    pad_agg = rows_agg * nw * ACHUNK - e
    zrows = 8 * pl.cdiv(n + 1, NS * 8)
    racc = zrows * NS
    sinks = racc - n

    src = edge_index[0]
    dst = edge_index[1]
    cyc_d = jnp.arange(pad_deg, dtype=jnp.int32)
    cyc_a = jnp.arange(pad_agg, dtype=jnp.int32)
    src_deg = jnp.concatenate([src, n + cyc_d % (brows * 128 - n)])
    src_agg = jnp.concatenate([src, n + cyc_a % 8]).reshape(-1, ACHUNK)
    dst2d = jnp.concatenate([dst, n + cyc_a % sinks]).reshape(-1, ACHUNK)

    zer_d = jnp.zeros((zrows, d), jnp.float32)
    zer_b = jnp.zeros((brows, 128), jnp.float32)
    iota_b = jnp.arange(brows, dtype=jnp.int32)

    deg2 = _deg_kernel(n, epw, brows)(src_deg, zer_b, iota_b)
    degf = deg2.reshape(NC, brows * 128)
    d0 = degf[0, :n, None]
    d1 = degf[1, :n, None]

    bn = 1000
    grid = (n // bn,)
    y = pl.pallas_call(
        _scale_body,
        grid=grid,
        in_specs=[
            pl.BlockSpec((bn, d), lambda i: (i, 0)),
            pl.BlockSpec((bn, 1), lambda i: (i, 0)),
            pl.BlockSpec((bn, 1), lambda i: (i, 0)),
        ],
        out_specs=pl.BlockSpec((bn, d), lambda i: (i, 0)),
        out_shape=jax.ShapeDtypeStruct((n, d), jnp.float32),
    )(x, d0, d1)

    y_g = jnp.concatenate([y, jnp.zeros((8, d), jnp.float32)], axis=0)
    s2 = _agg_kernel(n, d, rows_agg, zrows, racc)(y_g, src_agg, dst2d, zer_d)

    out = pl.pallas_call(
        _final_body,
        grid=grid,
        in_specs=[
            pl.BlockSpec((bn, d), lambda i: (i, 0)),
            pl.BlockSpec((bn, d), lambda i: (i, 0)),
            pl.BlockSpec((bn, d), lambda i: (i, 0)),
            pl.BlockSpec((bn, 1), lambda i: (i, 0)),
            pl.BlockSpec((bn, 1), lambda i: (i, 0)),
            pl.BlockSpec((d, d), lambda i: (0, 0)),
            pl.BlockSpec((d, d), lambda i: (0, 0)),
        ],
        out_specs=pl.BlockSpec((bn, d), lambda i: (i, 0)),
        out_shape=jax.ShapeDtypeStruct((n, d), jnp.float32),
    )(x, s2[0], s2[1], d0, d1, W1, W2)
    return out


# speedup vs baseline: 34.0149x; 1.0158x over previous
import dataclasses
import functools

import jax
import jax.numpy as jnp
from jax import lax
from jax.experimental import pallas as pl
from jax.experimental.pallas import tpu as pltpu
from jax.experimental.pallas import tpu_sc as plsc

NC = 2
NS = 16
L = 16
CHUNK = 128
ACHUNK = 128
ASLAB = 16


def _sc_mesh():
    return plsc.VectorSubcoreMesh(core_axis_name="c", subcore_axis_name="s")


def _sc_params():
    cp = pltpu.CompilerParams()
    if "needs_layout_passes" in pltpu.CompilerParams.__dataclass_fields__:
        cp = dataclasses.replace(cp, needs_layout_passes=False)
    return cp


def _deg_kernel(n, epw, brows):
    @functools.partial(
        pl.kernel,
        out_type=jax.ShapeDtypeStruct((NC, brows, 128), jnp.float32),
        mesh=_sc_mesh(),
        scratch_types=[
            pltpu.VMEM((epw,), jnp.int32),
            pltpu.VMEM((brows, 128), jnp.float32),
            pltpu.VMEM((brows,), jnp.int32),
            pltpu.VMEM_SHARED((brows, 128), jnp.float32),
        ],
        compiler_params=_sc_params(),
    )
    def deg_kernel(src_hbm, zer_hbm, iota_hbm, deg_out, idx_v, hist_v, iota_v,
                   acc):
        cid = lax.axis_index("c")
        sid = lax.axis_index("s")
        wid = cid * NS + sid

        @pl.when(sid < brows // 8)
        def _():
            pltpu.sync_copy(zer_hbm.at[pl.ds(0, 8), :],
                            acc.at[pl.ds(sid * 8, 8), :])

        pltpu.sync_copy(zer_hbm, hist_v)
        pltpu.sync_copy(iota_hbm, iota_v)
        pltpu.sync_copy(src_hbm.at[pl.ds(wid * epw, epw)], idx_v)
        ones16 = jnp.ones((16,), jnp.float32)

        @pl.loop(0, epw, step=16)
        def _(i):
            ix = idx_v[pl.ds(i, 16)]
            r = lax.shift_right_logical(ix, 7)
            c = lax.bitwise_and(ix, 127)
            plsc.addupdate_scatter(hist_v, [r, c], ones16)

        plsc.subcore_barrier()
        pltpu.sync_copy(hist_v, acc.at[iota_v], add=True)
        plsc.subcore_barrier()

        @pl.when(sid == 0)
        def _():
            pltpu.sync_copy(acc, deg_out.at[cid])

    return deg_kernel


def _agg_kernel(n, d, rows_per_w, zrows, racc):
    hrows = ASLAB
    nslab = rows_per_w // hrows

    @functools.partial(
        pl.kernel,
        out_type=jax.ShapeDtypeStruct((NC, n, d), jnp.float32),
        mesh=_sc_mesh(),
        scratch_types=[
            pltpu.VMEM((hrows, ACHUNK), jnp.int32),
            pltpu.VMEM((hrows, ACHUNK), jnp.int32),
            pltpu.VMEM((ACHUNK, d), jnp.float32),
            pltpu.VMEM((ACHUNK, d), jnp.float32),
            pltpu.VMEM_SHARED((racc, d), jnp.float32),
            pltpu.SemaphoreType.DMA,
            pltpu.SemaphoreType.DMA,
        ],
    )
    def agg_kernel(y_hbm, src_hbm, dst_hbm, zer_hbm, s_out, si_v, di_v,
                   rows_a, rows_b, acc, sem_a, sem_b):
        cid = lax.axis_index("c")
        sid = lax.axis_index("s")
        wid = cid * NS + sid
        pltpu.sync_copy(zer_hbm, acc.at[pl.ds(sid * zrows, zrows), :])
        plsc.subcore_barrier()

        for h in range(nslab):
            row0 = wid * rows_per_w + h * hrows
            pltpu.sync_copy(src_hbm.at[pl.ds(row0, hrows), :], si_v)
            pltpu.sync_copy(dst_hbm.at[pl.ds(row0, hrows), :], di_v)

            pltpu.async_copy(y_hbm.at[si_v.at[0]], rows_a, sem_a)

            @pl.loop(0, hrows // 2)
            def _(g):
                j = 2 * g
                pltpu.async_copy(y_hbm.at[si_v.at[j + 1]], rows_b, sem_b)
                pltpu.make_async_copy(y_hbm.at[si_v.at[j]], rows_a, sem_a).wait()
                pltpu.sync_copy(rows_a, acc.at[di_v.at[j]], add=True)
                jn = jnp.minimum(j + 2, hrows - 1)
                pltpu.async_copy(y_hbm.at[si_v.at[jn]], rows_a, sem_a)
                pltpu.make_async_copy(y_hbm.at[si_v.at[j + 1]], rows_b, sem_b).wait()
                pltpu.sync_copy(rows_b, acc.at[di_v.at[j + 1]], add=True)

            pltpu.make_async_copy(y_hbm.at[si_v.at[0]], rows_a, sem_a).wait()

        plsc.subcore_barrier()
        base = (n // (NS * 8)) * 8
        rem = n - base * NS
        pltpu.sync_copy(
            acc.at[pl.ds(sid * base, base), :],
            s_out.at[cid, pl.ds(sid * base, base), :],
        )
        if rem:
            @pl.when(sid == 0)
            def _():
                pltpu.sync_copy(
                    acc.at[pl.ds(base * NS, rem), :],
                    s_out.at[cid, pl.ds(base * NS, rem), :],
                )

    return agg_kernel


def _scale_body(x_ref, d0_ref, d1_ref, y_ref):
    deg = d0_ref[...] + d1_ref[...]
    r = lax.rsqrt(jnp.maximum(deg, 1.0))
    y_ref[...] = x_ref[...] * r


def _final_body(x_ref, s0_ref, s1_ref, d0_ref, d1_ref, w1_ref, w2_ref, o_ref):
    deg = d0_ref[...] + d1_ref[...]
    r = lax.rsqrt(jnp.maximum(deg, 1.0))
    t = (s0_ref[...] + s1_ref[...]) * r
    xv = x_ref[...]
    e = jnp.dot(xv + t, w1_ref[...], preferred_element_type=jnp.float32)
    e = e + jnp.dot(t * xv, w2_ref[...], preferred_element_type=jnp.float32)
    o_ref[...] = jnp.where(e >= 0, e, 0.01 * e)


@jax.jit
def kernel(x, edge_index, W1, W2):
    n, d = x.shape
    e = edge_index.shape[1]
    nw = NC * NS
    epw = 8 * pl.cdiv(e, nw * 8)
    brows = 8 * pl.cdiv(n + 1, 128 * 8)
    rows_agg = 8 * pl.cdiv(pl.cdiv(e, ACHUNK), nw * 8)
    pad_deg = epw * nw - e
    pad_agg = rows_agg * nw * ACHUNK - e
    zrows = 8 * pl.cdiv(n + 1, NS * 8)
    racc = zrows * NS
    sinks = racc - n

    src = edge_index[0]
    dst = edge_index[1]
    cyc_d = jnp.arange(pad_deg, dtype=jnp.int32)
    cyc_a = jnp.arange(pad_agg, dtype=jnp.int32)
    src_deg = jnp.concatenate([src, n + cyc_d % (brows * 128 - n)])
    src_agg = jnp.concatenate([src, n + cyc_a % 8]).reshape(-1, ACHUNK)
    dst2d = jnp.concatenate([dst, n + cyc_a % sinks]).reshape(-1, ACHUNK)

    zer_d = jnp.zeros((zrows, d), jnp.float32)
    zer_b = jnp.zeros((brows, 128), jnp.float32)
    iota_b = jnp.arange(brows, dtype=jnp.int32)

    deg2 = _deg_kernel(n, epw, brows)(src_deg, zer_b, iota_b)
    degf = deg2.reshape(NC, brows * 128)
    d0 = degf[0, :n, None]
    d1 = degf[1, :n, None]

    bn = 1000
    grid = (n // bn,)
    y = pl.pallas_call(
        _scale_body,
        grid=grid,
        in_specs=[
            pl.BlockSpec((bn, d), lambda i: (i, 0)),
            pl.BlockSpec((bn, 1), lambda i: (i, 0)),
            pl.BlockSpec((bn, 1), lambda i: (i, 0)),
        ],
        out_specs=pl.BlockSpec((bn, d), lambda i: (i, 0)),
        out_shape=jax.ShapeDtypeStruct((n + 8, d), jnp.float32),
    )(x, d0, d1)

    s2 = _agg_kernel(n, d, rows_agg, zrows, racc)(y, src_agg, dst2d, zer_d)

    out = pl.pallas_call(
        _final_body,
        grid=grid,
        in_specs=[
            pl.BlockSpec((bn, d), lambda i: (i, 0)),
            pl.BlockSpec((bn, d), lambda i: (i, 0)),
            pl.BlockSpec((bn, d), lambda i: (i, 0)),
            pl.BlockSpec((bn, 1), lambda i: (i, 0)),
            pl.BlockSpec((bn, 1), lambda i: (i, 0)),
            pl.BlockSpec((d, d), lambda i: (0, 0)),
            pl.BlockSpec((d, d), lambda i: (0, 0)),
        ],
        out_specs=pl.BlockSpec((bn, d), lambda i: (i, 0)),
        out_shape=jax.ShapeDtypeStruct((n, d), jnp.float32),
    )(x, s2[0], s2[1], d0, d1, W1, W2)
    return out
